# baseline (speedup 1.0000x reference)
import functools

import jax
import jax.numpy as jnp
from jax.experimental import pallas as pl
from jax.experimental.pallas import tpu as pltpu

INPUT_DIM = 2048
BOTTLENECK = 16384
K = 32


def _enc_body(wh_ref, wl_ref, xh_ref, xl_ref, b_ref, z1_ref):
    dn = (((1,), (1,)), ((), ()))
    z = jax.lax.dot_general(
        xh_ref[...], wh_ref[...], dimension_numbers=dn,
        preferred_element_type=jnp.float32)
    z += jax.lax.dot_general(
        xh_ref[...], wl_ref[...], dimension_numbers=dn,
        preferred_element_type=jnp.float32)
    z += jax.lax.dot_general(
        xl_ref[...], wh_ref[...], dimension_numbers=dn,
        preferred_element_type=jnp.float32)
    z1_ref[...] = z + b_ref[...]


def _encoder(xh, xl, Wh, Wl, b2d, tok_blk, bn_blk):
    n_tok = xh.shape[0]
    grid = (BOTTLENECK // bn_blk, n_tok // tok_blk)
    w_spec = pl.BlockSpec((bn_blk, INPUT_DIM), lambda j, i: (j, 0))
    x_spec = pl.BlockSpec((tok_blk, INPUT_DIM), lambda j, i: (i, 0))
    return pl.pallas_call(
        _enc_body,
        grid=grid,
        in_specs=[w_spec, w_spec, x_spec, x_spec,
                  pl.BlockSpec((1, bn_blk), lambda j, i: (0, j))],
        out_specs=pl.BlockSpec((tok_blk, bn_blk), lambda j, i: (i, j)),
        out_shape=jax.ShapeDtypeStruct((n_tok, BOTTLENECK), jnp.float32),
        compiler_params=pltpu.CompilerParams(
            dimension_semantics=("arbitrary", "arbitrary"),
        ),
    )(Wh, Wl, xh, xl, b2d)


def _topk_body(z1_ref, a1_ref, fs_ref):
    R = z1_ref.shape[0]
    N = z1_ref.shape[1]
    NEG = jnp.float32(-jnp.inf)

    fs_ref[:, : N // 2] = jnp.maximum(z1_ref[:, : N // 2], z1_ref[:, N // 2:])
    w = N // 4
    while w >= 128:
        fs_ref[:, :w] = jnp.maximum(fs_ref[:, :w], fs_ref[:, w:2 * w])
        w //= 2

    def extract(n_iter, width):
        def body(_, t):
            blk = fs_ref[:, :width]
            m = jnp.max(blk, axis=1, keepdims=True)
            fs_ref[:, :width] = jnp.where(blk >= m, NEG, blk)
            return m
        return jax.lax.fori_loop(
            0, n_iter, body, jnp.zeros((R, 1), jnp.float32))

    t_lo = extract(K + 1, 128)

    zlo = z1_ref[:, : N // 2]
    zhi = z1_ref[:, N // 2:]
    fs_ref[:, : N // 2] = jnp.maximum(
        jnp.where(zlo >= t_lo, zlo, NEG), jnp.where(zhi >= t_lo, zhi, NEG))
    w = N // 4
    while w >= 1024:
        fs_ref[:, :w] = jnp.maximum(fs_ref[:, :w], fs_ref[:, w:2 * w])
        w //= 2

    t1 = jnp.maximum(extract(K, 1024), t_lo)
    cnt1 = jnp.sum((z1_ref[...] >= t1).astype(jnp.float32),
                   axis=1, keepdims=True)

    kf = jnp.float32(K)

    def cond(carry):
        _, cnt = carry
        return jnp.any(cnt > kf)

    def body(carry):
        t, cnt = carry
        zz = z1_ref[...]
        active = cnt > kf
        m = jnp.min(jnp.where(zz >= t, zz, jnp.inf), axis=1, keepdims=True)
        m2 = jnp.min(jnp.where(zz > m, zz, jnp.inf), axis=1, keepdims=True)
        t_new = jnp.where(active, m2, t)
        cnt_new = jnp.sum((zz >= t_new).astype(jnp.float32),
                          axis=1, keepdims=True)
        return t_new, cnt_new

    thr, _ = jax.lax.while_loop(cond, body, (t1, cnt1))
    z = z1_ref[...]
    a1_ref[...] = jnp.where(z >= thr, z, 0.0).astype(jnp.bfloat16)


def _topk_mask(z1, tok_blk):
    n_tok = z1.shape[0]
    return pl.pallas_call(
        _topk_body,
        grid=(n_tok // tok_blk,),
        in_specs=[pl.BlockSpec((tok_blk, BOTTLENECK), lambda i: (i, 0))],
        out_specs=pl.BlockSpec((tok_blk, BOTTLENECK), lambda i: (i, 0)),
        out_shape=jax.ShapeDtypeStruct((n_tok, BOTTLENECK), jnp.bfloat16),
        scratch_shapes=[pltpu.VMEM((tok_blk, BOTTLENECK // 2), jnp.float32)],
        compiler_params=pltpu.CompilerParams(
            dimension_semantics=("arbitrary",),
        ),
    )(z1)


def _dec_body(a1_ref, w_ref, db_ref, z2_ref, acc_ref, *, n_kc):
    kc = pl.program_id(1)

    @pl.when(kc == 0)
    def _():
        acc_ref[...] = jnp.zeros_like(acc_ref)

    acc_ref[...] += jax.lax.dot_general(
        a1_ref[...], w_ref[...],
        dimension_numbers=(((1,), (0,)), ((), ())),
        preferred_element_type=jnp.float32,
    )

    @pl.when(kc == n_kc - 1)
    def _():
        z2_ref[...] = acc_ref[...] + db_ref[...]


def _decoder(a1, W, db2d, tok_blk, kc_blk):
    n_tok = a1.shape[0]
    n_kc = BOTTLENECK // kc_blk
    grid = (n_tok // tok_blk, n_kc)
    return pl.pallas_call(
        functools.partial(_dec_body, n_kc=n_kc),
        grid=grid,
        in_specs=[
            pl.BlockSpec((tok_blk, kc_blk), lambda i, k: (i, k)),
            pl.BlockSpec((kc_blk, INPUT_DIM), lambda i, k: (k, 0)),
            pl.BlockSpec((1, INPUT_DIM), lambda i, k: (0, 0)),
        ],
        out_specs=pl.BlockSpec((tok_blk, INPUT_DIM), lambda i, k: (i, 0)),
        out_shape=jax.ShapeDtypeStruct((n_tok, INPUT_DIM), jnp.float32),
        scratch_shapes=[pltpu.VMEM((tok_blk, INPUT_DIM), jnp.float32)],
        compiler_params=pltpu.CompilerParams(
            dimension_semantics=("arbitrary", "arbitrary"),
        ),
    )(a1, W, db2d)


def _split_bf16(a):
    hi = a.astype(jnp.bfloat16)
    lo = (a - hi.astype(jnp.float32)).astype(jnp.bfloat16)
    return hi, lo


def kernel(x, W, b, dec_bias):
    if x.ndim == 1:
        x = x[None, :]
    n_tok = x.shape[0]
    b2d = b.reshape(1, BOTTLENECK)
    db2d = dec_bias.reshape(1, INPUT_DIM)
    xh, xl = _split_bf16(x)
    Wh, Wl = _split_bf16(W)

    tok_blk_mm = min(512, n_tok)
    z1 = _encoder(xh, xl, Wh, Wl, b2d, tok_blk_mm, 1024)
    a1 = _topk_mask(z1, min(128, n_tok))
    z2 = _decoder(a1, Wh, db2d, tok_blk_mm, 2048)
    return z2
